# Initial kernel scaffold; baseline (speedup 1.0000x reference)
import jax
import jax.numpy as jnp
from jax.experimental import pallas as pl


def kernel(x, edge_index, edge_attr, batch, pos, params):
    raise NotImplementedError("write your pallas kernel here")


# speedup vs baseline: 2.8409x; 2.8409x over previous
import functools

import jax
import jax.numpy as jnp
from jax import lax
from jax.experimental import pallas as pl
from jax.experimental.pallas import tpu as pltpu
from jax.experimental.pallas import tpu_sc as plsc

N = 50000
E = 800000
S = 9
CO = 16
BATCH = 100
LIN_IN = 500 * 16
LIN_OUT = 8

NC = 2
NS = 16
NW = NC * NS

NA = 51200
EP = 819200
EPW = EP // NW
CH = 1024
NCHUNK = EPW // CH
RPC = CH // 128
ROWS_PW = EPW // 128
NPS = NA // NS

BE = 2048
BN = 2048

_mesh = functools.partial(
    plsc.VectorSubcoreMesh, core_axis_name="c", subcore_axis_name="s")
_sc_params = pltpu.CompilerParams(use_tc_tiling_on_sc=False)


@functools.lru_cache(maxsize=None)
def _make_gather(nrows, ci):
  @functools.partial(
      pl.kernel,
      out_type=jax.ShapeDtypeStruct((EP, ci), jnp.float32),
      mesh=_mesh(),
      scratch_types=[
          pltpu.VMEM((RPC, 128), jnp.int32),
          pltpu.VMEM((CH, ci), jnp.float32),
          pltpu.SemaphoreType.DMA,
      ],
      compiler_params=_sc_params,
  )
  def gather_k(table, src2, out, idx_v, rows_v, sem):
    cid = lax.axis_index("c")
    sid = lax.axis_index("s")
    wid = sid * NC + cid
    ebase = wid * EPW
    rbase = wid * ROWS_PW

    def step(i, carry):
      pltpu.sync_copy(src2.at[pl.ds(rbase + i * RPC, RPC)], idx_v)
      cps = [
          pltpu.async_copy(table.at[idx_v.at[j]],
                           rows_v.at[pl.ds(j * 128, 128)], sem)
          for j in range(RPC)
      ]
      for cp in cps:
        cp.wait()
      pltpu.sync_copy(rows_v, out.at[pl.ds(ebase + i * CH, CH)])
      return carry

    lax.fori_loop(0, NCHUNK, step, 0)

  return gather_k


def _scatter_body(msg, dst2, zeros, out, idx_v, msg_v, acc):
  cid = lax.axis_index("c")
  sid = lax.axis_index("s")
  wid = sid * NC + cid
  ebase = wid * EPW
  rbase = wid * ROWS_PW

  pltpu.sync_copy(zeros.at[pl.ds(sid * NPS, NPS)],
                  acc.at[pl.ds(sid * NPS, NPS)])
  plsc.subcore_barrier()

  def step(i, carry):
    pltpu.sync_copy(dst2.at[pl.ds(rbase + i * RPC, RPC)], idx_v)
    pltpu.sync_copy(msg.at[pl.ds(ebase + i * CH, CH)], msg_v)
    for j in range(RPC):
      pltpu.sync_copy(msg_v.at[pl.ds(j * 128, 128)],
                      acc.at[idx_v.at[j]], add=True)
    return carry

  lax.fori_loop(0, NCHUNK, step, 0)
  plsc.subcore_barrier()
  pltpu.sync_copy(acc.at[pl.ds(sid * NPS, NPS)],
                  out.at[pl.ds(cid * NA + sid * NPS, NPS)])


@functools.partial(
    pl.kernel,
    out_type=jax.ShapeDtypeStruct((2 * NA, CO), jnp.float32),
    mesh=_mesh(),
    scratch_types=[
        pltpu.VMEM((RPC, 128), jnp.int32),
        pltpu.VMEM((CH, CO), jnp.float32),
        pltpu.VMEM_SHARED((NA, CO), jnp.float32),
    ],
    compiler_params=_sc_params,
)
def _scatter_k(msg, dst2, zeros, out, idx_v, msg_v, acc):
  _scatter_body(msg, dst2, zeros, out, idx_v, msg_v, acc)


@functools.partial(
    pl.kernel,
    out_type=jax.ShapeDtypeStruct((4 * NA, CO), jnp.float32),
    mesh=_mesh(),
    scratch_types=[
        pltpu.VMEM((RPC, 128), jnp.int32),
        pltpu.VMEM((CH, CO), jnp.float32),
        pltpu.VMEM((128, CO), jnp.float32),
        pltpu.VMEM_SHARED((NA, CO), jnp.float32),
        pltpu.VMEM_SHARED((NA, CO), jnp.float32),
    ],
    compiler_params=_sc_params,
)
def _scatter_deg_k(msg, dst2, zeros, ones, out, idx_v, msg_v, ones_v,
                   acc, acc_d):
  cid = lax.axis_index("c")
  sid = lax.axis_index("s")
  wid = sid * NC + cid
  ebase = wid * EPW
  rbase = wid * ROWS_PW

  pltpu.sync_copy(zeros.at[pl.ds(sid * NPS, NPS)],
                  acc.at[pl.ds(sid * NPS, NPS)])
  pltpu.sync_copy(zeros.at[pl.ds(sid * NPS, NPS)],
                  acc_d.at[pl.ds(sid * NPS, NPS)])
  pltpu.sync_copy(ones, ones_v)
  plsc.subcore_barrier()

  def step(i, carry):
    pltpu.sync_copy(dst2.at[pl.ds(rbase + i * RPC, RPC)], idx_v)
    pltpu.sync_copy(msg.at[pl.ds(ebase + i * CH, CH)], msg_v)
    for j in range(RPC):
      pltpu.sync_copy(msg_v.at[pl.ds(j * 128, 128)],
                      acc.at[idx_v.at[j]], add=True)
      pltpu.sync_copy(ones_v, acc_d.at[idx_v.at[j]], add=True)
    return carry

  lax.fori_loop(0, NCHUNK, step, 0)
  plsc.subcore_barrier()
  pltpu.sync_copy(acc.at[pl.ds(sid * NPS, NPS)],
                  out.at[pl.ds(cid * NA + sid * NPS, NPS)])
  pltpu.sync_copy(acc_d.at[pl.ds(sid * NPS, NPS)],
                  out.at[pl.ds((2 + cid) * NA + sid * NPS, NPS)])


def _msg_body(pos_ref, xj_ref, w_ref, out_ref):
  pos = pos_ref[...]
  t = pos - jnp.floor(pos)
  t0 = t[:, 0:1]
  t1 = t[:, 1:2]

  def fs(tt):
    return (0.5 * tt * tt - tt + 0.5, -tt * tt + tt + 0.5, 0.5 * tt * tt)

  f0 = fs(t0)
  f1 = fs(t1)
  xj = xj_ref[...]
  acc = jnp.zeros((xj.shape[0], CO), jnp.float32)
  for a in range(3):
    for b in range(3):
      ws = w_ref[3 * a + b]
      acc = acc + (f1[a] * f0[b]) * jnp.dot(
          xj, ws, preferred_element_type=jnp.float32)
  out_ref[...] = acc


@functools.lru_cache(maxsize=None)
def _make_msg(ci):
  return pl.pallas_call(
      _msg_body,
      grid=(EP // BE,),
      in_specs=[
          pl.BlockSpec((BE, 2), lambda i: (i, 0)),
          pl.BlockSpec((BE, ci), lambda i: (i, 0)),
          pl.BlockSpec((S, ci, CO), lambda i: (0, 0, 0)),
      ],
      out_specs=pl.BlockSpec((BE, CO), lambda i: (i, 0)),
      out_shape=jax.ShapeDtypeStruct((EP, CO), jnp.float32),
  )


def _dinv_body(d0_ref, d1_ref, out_ref):
  deg = d0_ref[...] + d1_ref[...]
  out_ref[...] = 1.0 / jnp.maximum(deg, 1.0)


_dinv_k = pl.pallas_call(
    _dinv_body,
    grid=(NA // BN,),
    in_specs=[
        pl.BlockSpec((BN, CO), lambda i: (i + 2 * (NA // BN), 0)),
        pl.BlockSpec((BN, CO), lambda i: (i + 3 * (NA // BN), 0)),
    ],
    out_specs=pl.BlockSpec((BN, CO), lambda i: (i, 0)),
    out_shape=jax.ShapeDtypeStruct((NA, CO), jnp.float32),
)


def _comb_body(p0_ref, p1_ref, dinv_ref, x_ref, root_ref, b_ref, out_ref):
  agg = (p0_ref[...] + p1_ref[...]) * dinv_ref[...]
  o = agg + jnp.dot(x_ref[...], root_ref[...],
                    preferred_element_type=jnp.float32) + b_ref[...]
  out_ref[...] = jnp.minimum(jnp.maximum(o, 0.0), 6.0)


@functools.lru_cache(maxsize=None)
def _make_combine(ci):
  return pl.pallas_call(
      _comb_body,
      grid=(NA // BN,),
      in_specs=[
          pl.BlockSpec((BN, CO), lambda i: (i, 0)),
          pl.BlockSpec((BN, CO), lambda i: (i + NA // BN, 0)),
          pl.BlockSpec((BN, CO), lambda i: (i, 0)),
          pl.BlockSpec((BN, ci), lambda i: (i, 0)),
          pl.BlockSpec((ci, CO), lambda i: (0, 0)),
          pl.BlockSpec((1, CO), lambda i: (0, 0)),
      ],
      out_specs=pl.BlockSpec((BN, CO), lambda i: (i, 0)),
      out_shape=jax.ShapeDtypeStruct((NA, CO), jnp.float32),
  )


def _final_body(flat_ref, w_ref, b_ref, batch_ref, out_ref):
  m = jnp.max(batch_ref[...])
  delta = (m + 1 - BATCH).astype(jnp.float32)
  out_ref[...] = jnp.dot(flat_ref[...], w_ref[...],
                         preferred_element_type=jnp.float32) \
      + b_ref[...] + delta


_final_k = pl.pallas_call(
    _final_body,
    out_shape=jax.ShapeDtypeStruct((BATCH, LIN_OUT), jnp.float32),
)


def kernel(x, edge_index, edge_attr, batch, pos, params):
  del edge_attr
  f32 = jnp.float32
  src = edge_index[0]
  dst = edge_index[1]
  pad_e = EP - E
  src2 = jnp.concatenate(
      [src, jnp.zeros((pad_e,), jnp.int32)]).reshape(EP // 128, 128)
  dst2 = jnp.concatenate(
      [dst, jnp.full((pad_e,), N, jnp.int32)]).reshape(EP // 128, 128)
  posp = jnp.concatenate([pos, jnp.zeros((pad_e, 2), f32)], axis=0)
  zeros_n = jnp.zeros((NA, CO), f32)
  ones_sc = jnp.ones((128, CO), f32)

  xpad = jnp.zeros((NA, CO), f32).at[:N].set(x)

  def conv(h, ci, l, dinv):
    w = params['conv%d_w' % l]
    root = params['conv%d_root' % l]
    b = params['conv%d_b' % l].reshape(1, CO)
    xj = _make_gather(NA, ci)(h, src2)
    msg = _make_msg(ci)(posp, xj, w)
    p = _scatter_k(msg, dst2, zeros_n)
    return _make_combine(ci)(p, p, dinv, h, root, b)

  h = xpad
  xj = _make_gather(NA, CO)(h, src2)
  msg = _make_msg(CO)(posp, xj, params['conv0_w'])
  p = _scatter_deg_k(msg, dst2, zeros_n, ones_sc)
  dinv = _dinv_k(p, p)
  h = _make_combine(CO)(p, p, dinv, h,
                        params['conv0_root'],
                        params['conv0_b'].reshape(1, CO))
  outs = [xpad, h]
  for l in range(1, 3):
    h = conv(h, CO, l, dinv)
    outs.append(h)

  dcat = jnp.concatenate([outs[3], outs[2]], axis=1)
  d = conv(dcat, 2 * CO, 3, dinv)
  dcat = jnp.concatenate([d, outs[1]], axis=1)
  d = conv(dcat, 2 * CO, 4, dinv)

  flat = d[:N].reshape(BATCH, LIN_IN)
  batch2 = batch.reshape(BATCH, N // BATCH)
  return _final_k(flat, params['lin_w'],
                  params['lin_b'].reshape(1, LIN_OUT), batch2)
